# Initial kernel scaffold; baseline (speedup 1.0000x reference)
import jax
import jax.numpy as jnp
from jax.experimental import pallas as pl


def kernel(input, A, w):
    raise NotImplementedError("write your pallas kernel here")


# speedup vs baseline: 2.9479x; 2.9479x over previous
import functools

import jax
import jax.numpy as jnp
from jax.experimental import pallas as pl
from jax.experimental.pallas import tpu as pltpu


def _bitonic_sort_axis0(s):
    n = s.shape[0]
    r = jax.lax.broadcasted_iota(jnp.int32, (n, 1), 0)
    k = 2
    while k <= n:
        j = k // 2
        while j >= 1:
            bitj = (r & j) != 0
            bitk = (r & k) != 0
            keep_min = bitj == bitk
            up = jnp.roll(s, -j, axis=0)
            down = jnp.roll(s, j, axis=0)
            partner = jnp.where(bitj, down, up)
            mn = jnp.minimum(s, partner)
            mx = jnp.maximum(s, partner)
            s = jnp.where(keep_min, mn, mx)
            j //= 2
        k *= 2
    return s


def _body(x_ref, a_ref, w_ref, o_ref):
    t = pl.program_id(1)
    tile = a_ref.shape[1]
    xb = x_ref[0]
    a = a_ref[...]
    p = jax.lax.dot_general(
        xb, a, (((0,), (0,)), ((), ())),
        preferred_element_type=jnp.float32,
    )
    p = _bitonic_sort_axis0(p)
    wb = w_ref[0]
    o_ref[0, 0, pl.ds(t * tile, tile)] = jnp.sum(p * wb, axis=0)


@jax.jit
def kernel(input, A, w):
    B, D, N = input.shape
    D_OUT = A.shape[1]
    T = min(256, D_OUT)
    grid = (B, D_OUT // T)
    return pl.pallas_call(
        _body,
        grid=grid,
        in_specs=[
            pl.BlockSpec((1, D, N), lambda b, t: (b, 0, 0)),
            pl.BlockSpec((D, T), lambda b, t: (0, t)),
            pl.BlockSpec((1, N, T), lambda b, t: (0, 0, t)),
        ],
        out_specs=pl.BlockSpec((1, 1, D_OUT), lambda b, t: (b, 0, 0)),
        out_shape=jax.ShapeDtypeStruct((B, 1, D_OUT), jnp.float32),
    )(input, A, w)[:, 0, :]


# speedup vs baseline: 3.5759x; 1.2131x over previous
import jax
import jax.numpy as jnp
from jax.experimental import pallas as pl
from jax.experimental.pallas import tpu as pltpu

N = 1024
CHUNK = 128


def _cex_reshape(s, K, j, r0, static_dir):
    R, L = s.shape
    m = R // (2 * j)
    s4 = s.reshape(m, 2, j, L)
    a = s4[:, 0:1]
    b = s4[:, 1:2]
    mn = jnp.minimum(a, b)
    mx = jnp.maximum(a, b)
    if static_dir is None:
        blk = jax.lax.broadcasted_iota(jnp.int32, (m, 1, j, 1), 0)
        t = jax.lax.broadcasted_iota(jnp.int32, (m, 1, j, 1), 2)
        row = r0 + blk * (2 * j) + t
        asc = (row & K) == 0
        lo = jnp.where(asc, mn, mx)
        hi = jnp.where(asc, mx, mn)
    elif static_dir:
        lo, hi = mn, mx
    else:
        lo, hi = mx, mn
    return jnp.concatenate([lo, hi], axis=1).reshape(R, L)


def _cex_roll(s, K, j, r0, static_dir):
    R, L = s.shape
    ii = r0 + jax.lax.broadcasted_iota(jnp.int32, (R, 1), 0)
    bitj = (ii & j) != 0
    up = jnp.roll(s, -j, axis=0)
    down = jnp.roll(s, j, axis=0)
    partner = jnp.where(bitj, down, up)
    mn = jnp.minimum(s, partner)
    mx = jnp.maximum(s, partner)
    if static_dir is None:
        keep_min = bitj == ((ii & K) != 0)
        return jnp.where(keep_min, mn, mx)
    if static_dir:
        return jnp.where(bitj, mx, mn)
    return jnp.where(bitj, mn, mx)


def _cex(s, K, j, r0, static_dir):
    if j >= 8:
        return _cex_reshape(s, K, j, r0, static_dir)
    return _cex_roll(s, K, j, r0, static_dir)


def _sub_js(K, j_max):
    j = j_max
    while j >= 1:
        yield j
        j //= 2


def _merge_tail(s_ref, K, j_max, chunk):
    n = s_ref.shape[0]
    for c in range(n // chunk):
        r0 = c * chunk
        static_dir = (r0 & K) == 0
        s = s_ref[pl.ds(r0, chunk), :]
        for j in _sub_js(K, j_max):
            s = _cex(s, K, j, r0, static_dir)
        s_ref[pl.ds(r0, chunk), :] = s


def _merge_strided(s_ref, K, j_list, stride, nslice):
    for base in range(0, N, stride * nslice):
        asc = (base & K) == 0
        for r0 in range(0, stride, 8):
            q = [s_ref[pl.ds(base + r0 + i * stride, 8), :] for i in range(nslice)]
            for j in j_list:
                d = j // stride
                for i in range(nslice):
                    if i & d:
                        continue
                    a, b = q[i], q[i + d]
                    mn = jnp.minimum(a, b)
                    mx = jnp.maximum(a, b)
                    q[i], q[i + d] = (mn, mx) if asc else (mx, mn)
            for i in range(nslice):
                s_ref[pl.ds(base + r0 + i * stride, 8), :] = q[i]


def _bitonic_sort_ref(s_ref):
    n = s_ref.shape[0]
    for c in range(n // CHUNK):
        r0 = c * CHUNK
        s = s_ref[pl.ds(r0, CHUNK), :]
        K = 2
        while K <= CHUNK:
            sd = ((r0 & K) == 0) if K >= CHUNK else None
            for j in _sub_js(K, K // 2):
                s = _cex(s, K, j, r0, sd)
            K *= 2
        s_ref[pl.ds(r0, CHUNK), :] = s
    for c in range(n // 256):
        r0 = c * 256
        s = s_ref[pl.ds(r0, 256), :]
        for j in _sub_js(256, 128):
            s = _cex(s, 256, j, r0, (r0 & 256) == 0)
        s_ref[pl.ds(r0, 256), :] = s
    _merge_strided(s_ref, 512, [256, 128], 128, 4)
    _merge_tail(s_ref, 512, 64, CHUNK)
    _merge_strided(s_ref, 1024, [512, 256, 128], 128, 8)
    _merge_tail(s_ref, 1024, 64, CHUNK)


def _body(x_ref, a_ref, w_ref, o_ref, s_ref):
    t = pl.program_id(1)
    tile = a_ref.shape[1]
    xb = x_ref[0]
    a = a_ref[...]
    s_ref[...] = jax.lax.dot_general(
        xb, a, (((0,), (0,)), ((), ())),
        preferred_element_type=jnp.float32,
    )
    _bitonic_sort_ref(s_ref)
    wb = w_ref[0]
    o_ref[0, 0, pl.ds(t * tile, tile)] = jnp.sum(s_ref[...] * wb, axis=0)


@jax.jit
def kernel(input, A, w):
    B, D, n = input.shape
    D_OUT = A.shape[1]
    T = 128
    grid = (B, D_OUT // T)
    return pl.pallas_call(
        _body,
        grid=grid,
        in_specs=[
            pl.BlockSpec((1, D, n), lambda b, t: (b, 0, 0)),
            pl.BlockSpec((D, T), lambda b, t: (0, t)),
            pl.BlockSpec((1, n, T), lambda b, t: (0, 0, t)),
        ],
        out_specs=pl.BlockSpec((1, 1, D_OUT), lambda b, t: (b, 0, 0)),
        out_shape=jax.ShapeDtypeStruct((B, 1, D_OUT), jnp.float32),
        scratch_shapes=[pltpu.VMEM((n, T), jnp.float32)],
    )(input, A, w)[:, 0, :]


# speedup vs baseline: 7.2978x; 2.0408x over previous
import jax
import jax.numpy as jnp
from jax.experimental import pallas as pl
from jax.experimental.pallas import tpu as pltpu

N = 1024


def _swap_halves(s, dp):
    R, L = s.shape
    s3 = s.reshape(R // (2 * dp), 2 * dp, L)
    p3 = jnp.concatenate([s3[:, dp:], s3[:, :dp]], axis=1)
    return p3.reshape(R, L)


def _cex_small(s, dp):
    partner = _swap_halves(s, dp)
    mn = jnp.minimum(s, partner)
    mx = jnp.maximum(s, partner)
    ii = jax.lax.broadcasted_iota(jnp.int32, (s.shape[0], 1), 0)
    return jnp.where((ii & dp) == 0, mn, mx)


def _cex_big(s, dp):
    R, L = s.shape
    m = R // (2 * dp)
    s4 = s.reshape(m, 2, dp, L)
    a = s4[:, 0:1]
    b = s4[:, 1:2]
    mn = jnp.minimum(a, b)
    mx = jnp.maximum(a, b)
    return jnp.concatenate([mn, mx], axis=1).reshape(R, L)


def _cex(s, dp):
    return _cex_small(s, dp) if dp < 8 else _cex_big(s, dp)


def _first_levels_pass(s_ref):
    for r0 in range(0, 64, 8):
        q = [s_ref[pl.ds(r0 + 64 * i, 8), :] for i in range(16)]

        def flip(pred):
            for i in range(16):
                if pred(i):
                    q[i] = -q[i]

        def cex_slices(dp):
            dd = dp // 64
            for i in range(16):
                if i & dd:
                    continue
                a, b = q[i], q[i + dd]
                q[i] = jnp.minimum(a, b)
                q[i + dd] = jnp.maximum(a, b)

        flip(lambda i: i & 4)
        cex_slices(512)
        flip(lambda i: bool(i & 4) != bool(i & 2))
        cex_slices(256)
        cex_slices(512)
        flip(lambda i: bool(i & 2) != bool(i & 1))
        cex_slices(128)
        cex_slices(256)
        cex_slices(512)
        flip(lambda i: bool(i & 1) != bool(r0 & 32))
        cex_slices(64)
        cex_slices(128)
        cex_slices(256)
        cex_slices(512)
        for i in range(16):
            s_ref[pl.ds(r0 + 64 * i, 8), :] = q[i]


def _chunk_pass(s_ref, K):
    b_hi = 1024 // K
    b_lo = 512 // K
    for c in range(N // 64):
        r0 = c * 64
        s = s_ref[pl.ds(r0, 64), :]
        ii = r0 + jax.lax.broadcasted_iota(jnp.int32, (64, 1), 0)
        m = (ii & b_hi) != 0
        if b_lo:
            m = m != ((ii & b_lo) != 0)
        s = jnp.where(m, -s, s)
        dp = 1024 // K
        while dp <= 32:
            s = _cex(s, dp)
            dp *= 2
        s_ref[pl.ds(r0, 64), :] = s


def _strided_pass(s_ref):
    for r0 in range(0, 64, 8):
        q = [s_ref[pl.ds(r0 + 64 * i, 8), :] for i in range(16)]
        for dd in (1, 2, 4, 8):
            for i in range(16):
                if i & dd:
                    continue
                a, b = q[i], q[i + dd]
                q[i] = jnp.minimum(a, b)
                q[i + dd] = jnp.maximum(a, b)
        for i in range(16):
            s_ref[pl.ds(r0 + 64 * i, 8), :] = q[i]


def _bitonic_sort_ref(s_ref):
    _first_levels_pass(s_ref)
    for K in (32, 64, 128, 256, 512, 1024):
        _chunk_pass(s_ref, K)
        _strided_pass(s_ref)


def _body(x_ref, a_ref, w_ref, o_ref, s_ref):
    t = pl.program_id(1)
    tile = a_ref.shape[1]
    xb = x_ref[0]
    a = a_ref[...]
    s_ref[...] = jax.lax.dot_general(
        xb, a, (((0,), (0,)), ((), ())),
        preferred_element_type=jnp.float32,
    )
    _bitonic_sort_ref(s_ref)
    wb = w_ref[0]
    o_ref[0, 0, pl.ds(t * tile, tile)] = jnp.sum(s_ref[...] * wb, axis=0)


def _bitrev_perm(n):
    bits = n.bit_length() - 1
    return [int(format(i, f"0{bits}b")[::-1], 2) for i in range(n)]


@jax.jit
def kernel(input, A, w):
    B, D, n = input.shape
    D_OUT = A.shape[1]
    T = 128
    grid = (B, D_OUT // T)
    wp = jnp.take(w, jnp.array(_bitrev_perm(n), dtype=jnp.int32), axis=1)
    return pl.pallas_call(
        _body,
        grid=grid,
        in_specs=[
            pl.BlockSpec((1, D, n), lambda b, t: (b, 0, 0)),
            pl.BlockSpec((D, T), lambda b, t: (0, t)),
            pl.BlockSpec((1, n, T), lambda b, t: (0, 0, t)),
        ],
        out_specs=pl.BlockSpec((1, 1, D_OUT), lambda b, t: (b, 0, 0)),
        out_shape=jax.ShapeDtypeStruct((B, 1, D_OUT), jnp.float32),
        scratch_shapes=[pltpu.VMEM((n, T), jnp.float32)],
    )(input, A, wp)[:, 0, :]
